# Initial kernel scaffold; baseline (speedup 1.0000x reference)
import jax
import jax.numpy as jnp
from jax.experimental import pallas as pl


def kernel(x, edge_index, W1, b1, W2, b2):
    raise NotImplementedError("write your pallas kernel here")


# speedup vs baseline: 3.0894x; 3.0894x over previous
import functools

import jax
import jax.numpy as jnp
from jax.experimental import pallas as pl

N = 10000
F_IN = 128
HID = 16
C = 40


def _dense1_body(x_ref, w1_ref, deg_ref, gs1_ref, dinv_ref):
    deg = deg_ref[...]
    dinv = jax.lax.rsqrt(deg)
    g = jnp.dot(x_ref[...], w1_ref[...], preferred_element_type=jnp.float32)
    gs1_ref[...] = dinv * g
    dinv_ref[...] = dinv


def _dense2_body(seg_ref, gs1_ref, dinv_ref, b1_ref, w2_ref, gs2_ref):
    dinv = dinv_ref[...]
    h1 = jnp.maximum(dinv * (seg_ref[...] + gs1_ref[...]) + b1_ref[...], 0.0)
    g2 = jnp.dot(h1, w2_ref[...], preferred_element_type=jnp.float32)
    gs2_ref[...] = dinv * g2


def _dense3_body(seg_ref, gs2_ref, dinv_ref, b2_ref, out_ref):
    o = dinv_ref[...] * (seg_ref[...] + gs2_ref[...]) + b2_ref[...]
    m = jnp.max(o, axis=1, keepdims=True)
    lse = jnp.log(jnp.sum(jnp.exp(o - m), axis=1, keepdims=True)) + m
    out_ref[...] = o - lse


def _dense1(x, W1, deg):
    return pl.pallas_call(
        _dense1_body,
        out_shape=(
            jax.ShapeDtypeStruct((N, HID), jnp.float32),
            jax.ShapeDtypeStruct((N, 1), jnp.float32),
        ),
    )(x, W1, deg)


def _dense2(seg1, gs1, dinv, b1, W2):
    return pl.pallas_call(
        _dense2_body,
        out_shape=jax.ShapeDtypeStruct((N, C), jnp.float32),
    )(seg1, gs1, dinv, b1.reshape(1, HID), W2)


def _dense3(seg2, gs2, dinv, b2):
    return pl.pallas_call(
        _dense3_body,
        out_shape=jax.ShapeDtypeStruct((N, C), jnp.float32),
    )(seg2, gs2, dinv, b2.reshape(1, C))


def _segsum(vals_src, dst, width):
    return jax.ops.segment_sum(vals_src, dst, num_segments=N)


def kernel(x, edge_index, W1, b1, W2, b2):
    src = edge_index[0]
    dst = edge_index[1]
    deg = (1.0 + jax.ops.segment_sum(
        jnp.ones_like(dst, dtype=jnp.float32), dst, num_segments=N)
    ).reshape(N, 1)
    gs1, dinv = _dense1(x, W1, deg)
    seg1 = _segsum(gs1[src], dst, HID)
    gs2 = _dense2(seg1, gs1, dinv, b1, W2)
    seg2 = _segsum(gs2[src], dst, C)
    return _dense3(seg2, gs2, dinv, b2)


# speedup vs baseline: 40.7619x; 13.1943x over previous
import functools

import jax
import jax.numpy as jnp
from jax import lax
from jax.experimental import pallas as pl
from jax.experimental.pallas import tpu as pltpu
from jax.experimental.pallas import tpu_sc as plsc

N = 10000
F_IN = 128
HID = 16
C = 40
CP = 48
E = 320000

NC = 2
NS = 16
NT = NC * NS
EPT = E // NT
K = 100
NCH = EPT // K
WIN = 10
NB = NCH // WIN
NPAD = 10240
RPT = NPAD // NS

_MESH = plsc.VectorSubcoreMesh(core_axis_name="c", subcore_axis_name="s")


def _deg_body(dst_hbm, zeros_hbm, out_hbm, dstv, onesv, acc, sem_s):
    cid = lax.axis_index("c")
    sid = lax.axis_index("s")
    tile = cid * NS + sid
    row0 = sid * RPT
    pltpu.sync_copy(dst_hbm.at[tile], dstv)
    pltpu.sync_copy(zeros_hbm.at[pl.ds(row0, RPT)], acc.at[pl.ds(row0, RPT)])
    for r in range(K):
        onesv[r] = jnp.ones((HID,), jnp.float32)
    plsc.subcore_barrier()

    def window(b, carry):
        hs = []
        for i in range(WIN):
            j = b * WIN + i
            hs.append(pltpu.async_copy(onesv, acc.at[dstv.at[j]], sem_s,
                                       add=True))
        for h in hs:
            h.wait()
        return carry

    lax.fori_loop(0, NB, window, 0)
    plsc.subcore_barrier()
    pltpu.sync_copy(acc.at[pl.ds(row0, RPT)],
                    out_hbm.at[cid, pl.ds(row0, RPT)])


@functools.partial(
    pl.kernel,
    out_type=jax.ShapeDtypeStruct((NC, NPAD, HID), jnp.float32),
    mesh=_MESH,
    compiler_params=pltpu.CompilerParams(use_tc_tiling_on_sc=False),
    scratch_types=[
        pltpu.VMEM((NCH, K), jnp.int32),
        pltpu.VMEM((K, HID), jnp.float32),
        pltpu.VMEM_SHARED((NPAD, HID), jnp.float32),
        pltpu.SemaphoreType.DMA,
    ],
)
def _deg_pass(dst_hbm, zeros_hbm, out_hbm, dstv, onesv, acc, sem_s):
    _deg_body(dst_hbm, zeros_hbm, out_hbm, dstv, onesv, acc, sem_s)


def _segsum_body(table_hbm, src_hbm, dst_hbm, zeros_hbm, out_hbm,
                 srcv, dstv, rows, acc, sem_g, sem_s):
    cid = lax.axis_index("c")
    sid = lax.axis_index("s")
    tile = cid * NS + sid
    row0 = sid * RPT
    pltpu.sync_copy(src_hbm.at[tile], srcv)
    pltpu.sync_copy(dst_hbm.at[tile], dstv)
    pltpu.sync_copy(zeros_hbm.at[pl.ds(row0, RPT)], acc.at[pl.ds(row0, RPT)])
    plsc.subcore_barrier()

    def window(b, carry):
        hs = []
        for i in range(WIN):
            j = b * WIN + i
            hs.append(pltpu.async_copy(table_hbm.at[srcv.at[j]], rows.at[i],
                                       sem_g))
        for h in hs:
            h.wait()
        ss = []
        for i in range(WIN):
            j = b * WIN + i
            ss.append(pltpu.async_copy(rows.at[i], acc.at[dstv.at[j]], sem_s,
                                       add=True))
        for s in ss:
            s.wait()
        return carry

    lax.fori_loop(0, NB, window, 0)
    plsc.subcore_barrier()
    pltpu.sync_copy(acc.at[pl.ds(row0, RPT)],
                    out_hbm.at[cid, pl.ds(row0, RPT)])


def _make_segsum(F):
    @functools.partial(
        pl.kernel,
        out_type=jax.ShapeDtypeStruct((NC, NPAD, F), jnp.float32),
        mesh=_MESH,
        compiler_params=pltpu.CompilerParams(use_tc_tiling_on_sc=False),
        scratch_types=[
            pltpu.VMEM((NCH, K), jnp.int32),
            pltpu.VMEM((NCH, K), jnp.int32),
            pltpu.VMEM((WIN, K, F), jnp.float32),
            pltpu.VMEM_SHARED((NPAD, F), jnp.float32),
            pltpu.SemaphoreType.DMA,
            pltpu.SemaphoreType.DMA,
        ],
    )
    def segsum(table_hbm, src_hbm, dst_hbm, zeros_hbm, out_hbm,
               srcv, dstv, rows, acc, sem_g, sem_s):
        _segsum_body(table_hbm, src_hbm, dst_hbm, zeros_hbm, out_hbm,
                     srcv, dstv, rows, acc, sem_g, sem_s)

    return segsum


_segsum16 = _make_segsum(HID)
_segsum48 = _make_segsum(CP)


def _dense1_body(x_ref, w1_ref, degp_ref, gs1_ref, dinv_ref):
    degp = degp_ref[...]
    deg = 1.0 + degp[0, :, 0:1] + degp[1, :, 0:1]
    dinv = lax.rsqrt(deg)
    g = jnp.dot(x_ref[...], w1_ref[...], preferred_element_type=jnp.float32)
    gs1_ref[...] = dinv * g
    dinv_ref[...] = dinv


def _dense2_body(segp_ref, gs1_ref, dinv_ref, b1_ref, w2_ref, gs2_ref):
    dinv = dinv_ref[...]
    segp = segp_ref[...]
    seg = segp[0] + segp[1]
    h1 = jnp.maximum(dinv * (seg + gs1_ref[...]) + b1_ref[...], 0.0)
    g2 = jnp.dot(h1, w2_ref[...], preferred_element_type=jnp.float32)
    gs2_ref[...] = dinv * g2


def _dense3_body(segp_ref, gs2_ref, dinv_ref, b2_ref, out_ref):
    segp = segp_ref[...]
    o = dinv_ref[...] * (segp[0] + segp[1] + gs2_ref[...]) + b2_ref[...]
    col = lax.broadcasted_iota(jnp.int32, (N, CP), 1)
    om = jnp.where(col < C, o, -1e30)
    m = jnp.max(om, axis=1, keepdims=True)
    lse = jnp.log(jnp.sum(jnp.exp(om - m), axis=1, keepdims=True)) + m
    out_ref[...] = (o - lse)[:, :C]


def _dense1(x, W1, degp):
    return pl.pallas_call(
        _dense1_body,
        out_shape=(
            jax.ShapeDtypeStruct((N, HID), jnp.float32),
            jax.ShapeDtypeStruct((N, 1), jnp.float32),
        ),
    )(x, W1, degp)


def _dense2(segp, gs1, dinv, b1, W2p):
    return pl.pallas_call(
        _dense2_body,
        out_shape=jax.ShapeDtypeStruct((N, CP), jnp.float32),
    )(segp, gs1, dinv, b1.reshape(1, HID), W2p)


def _dense3(segp, gs2, dinv, b2p):
    return pl.pallas_call(
        _dense3_body,
        out_shape=jax.ShapeDtypeStruct((N, C), jnp.float32),
    )(segp, gs2, dinv, b2p.reshape(1, CP))


def kernel(x, edge_index, W1, b1, W2, b2):
    src = edge_index[0].reshape(NT, NCH, K)
    dst = edge_index[1].reshape(NT, NCH, K)
    zeros16 = jnp.zeros((NPAD, HID), jnp.float32)
    zeros48 = jnp.zeros((NPAD, CP), jnp.float32)
    W2p = jnp.pad(W2, ((0, 0), (0, CP - C)))
    b2p = jnp.pad(b2, (0, CP - C))

    degp = _deg_pass(dst, zeros16)[:, :N]
    gs1, dinv = _dense1(x, W1, degp)
    segp1 = _segsum16(gs1, src, dst, zeros16)[:, :N]
    gs2 = _dense2(segp1, gs1, dinv, b1, W2p)
    segp2 = _segsum48(gs2, src, dst, zeros48)[:, :N]
    return _dense3(segp2, gs2, dinv, b2p)


# speedup vs baseline: 49.6249x; 1.2174x over previous
import functools

import jax
import jax.numpy as jnp
from jax import lax
from jax.experimental import pallas as pl
from jax.experimental.pallas import tpu as pltpu
from jax.experimental.pallas import tpu_sc as plsc

N = 10000
F_IN = 128
HID = 16
C = 40
CP = 48
E = 320000

NC = 2
NS = 16
NT = NC * NS
EPT = E // NT
K = 80
NCH = EPT // K
WIN = 5
NB = NCH // WIN
GROW = WIN * K
NPAD = 10240
RPT = NPAD // NS

_MESH = plsc.VectorSubcoreMesh(core_axis_name="c", subcore_axis_name="s")
_SC_PARAMS = pltpu.CompilerParams(use_tc_tiling_on_sc=False)


def _deg_body(dst_hbm, zeros_hbm, out_hbm, dstv, onesv, acc, sem_s):
    cid = lax.axis_index("c")
    sid = lax.axis_index("s")
    tile = cid * NS + sid
    row0 = sid * RPT
    pltpu.sync_copy(dst_hbm.at[tile], dstv)
    pltpu.sync_copy(zeros_hbm, acc.at[pl.ds(row0, RPT)])
    for r in range(K):
        onesv[r] = jnp.ones((HID,), jnp.float32)
    plsc.subcore_barrier()

    DEPTH = 16
    drain1 = out_hbm.at[0, pl.ds(0, K)]

    def chunk(j, carry):
        pltpu.async_copy(onesv, acc.at[dstv.at[j]], sem_s, add=True)

        @pl.when(j >= DEPTH)
        def _():
            pltpu.make_async_copy(drain1, onesv, sem_s).wait()

        return carry

    lax.fori_loop(0, NCH, chunk, 0)
    for _ in range(DEPTH):
        pltpu.make_async_copy(drain1, onesv, sem_s).wait()
    plsc.subcore_barrier()
    pltpu.sync_copy(acc.at[pl.ds(row0, RPT)],
                    out_hbm.at[cid, pl.ds(row0, RPT)])


@functools.partial(
    pl.kernel,
    out_type=jax.ShapeDtypeStruct((NC, NPAD, HID), jnp.float32),
    mesh=_MESH,
    compiler_params=_SC_PARAMS,
    scratch_types=[
        pltpu.VMEM((NCH, K), jnp.int32),
        pltpu.VMEM((K, HID), jnp.float32),
        pltpu.VMEM_SHARED((NPAD, HID), jnp.float32),
        pltpu.SemaphoreType.DMA,
    ],
)
def _deg_pass(dst_hbm, zeros_hbm, out_hbm, dstv, onesv, acc, sem_s):
    _deg_body(dst_hbm, zeros_hbm, out_hbm, dstv, onesv, acc, sem_s)


def _segsum_body(F, table_hbm, src_hbm, dst_hbm, zeros_hbm, out_hbm,
                 srcv, dstv, rows0, rows1, acc, sg0, sg1, ss0, ss1):
    cid = lax.axis_index("c")
    sid = lax.axis_index("s")
    tile = cid * NS + sid
    row0 = sid * RPT
    pltpu.sync_copy(src_hbm.at[tile], srcv)
    pltpu.sync_copy(dst_hbm.at[tile], dstv)
    pltpu.sync_copy(zeros_hbm, acc.at[pl.ds(row0, RPT)])
    plsc.subcore_barrier()

    drain_src = out_hbm.at[0, pl.ds(0, GROW)]

    def fire_gathers(b, rows_c, sg_c):
        for i in range(WIN):
            j = b * WIN + i
            pltpu.async_copy(table_hbm.at[srcv.at[j]],
                             rows_c.at[pl.ds(i * K, K)], sg_c)

    def fire_scatters(b, rows_c, ss_c):
        for i in range(WIN):
            j = b * WIN + i
            pltpu.async_copy(rows_c.at[pl.ds(i * K, K)],
                             acc.at[dstv.at[j]], ss_c, add=True)

    def step(b, rows_c, sg_c, ss_c, rows_o, sg_o, ss_o):
        @pl.when(b >= 2)
        def _():
            pltpu.make_async_copy(drain_src, rows_c, ss_c).wait()

        fire_gathers(b, rows_c, sg_c)

        @pl.when(b >= 1)
        def _():
            pltpu.make_async_copy(drain_src, rows_o, sg_o).wait()
            fire_scatters(b - 1, rows_o, ss_o)

    def body(b, carry):
        @pl.when(b % 2 == 0)
        def _():
            step(b, rows0, sg0, ss0, rows1, sg1, ss1)

        @pl.when(b % 2 == 1)
        def _():
            step(b, rows1, sg1, ss1, rows0, sg0, ss0)

        return carry

    lax.fori_loop(0, NB, body, 0)
    pltpu.make_async_copy(drain_src, rows0, sg0).wait()
    fire_scatters(NB - 1, rows0, ss0)
    pltpu.make_async_copy(drain_src, rows1, ss1).wait()
    pltpu.make_async_copy(drain_src, rows0, ss0).wait()
    plsc.subcore_barrier()
    pltpu.sync_copy(acc.at[pl.ds(row0, RPT)],
                    out_hbm.at[cid, pl.ds(row0, RPT)])


def _make_segsum(F):
    @functools.partial(
        pl.kernel,
        out_type=jax.ShapeDtypeStruct((NC, NPAD, F), jnp.float32),
        mesh=_MESH,
        compiler_params=_SC_PARAMS,
        scratch_types=[
            pltpu.VMEM((NCH, K), jnp.int32),
            pltpu.VMEM((NCH, K), jnp.int32),
            pltpu.VMEM((GROW, F), jnp.float32),
            pltpu.VMEM((GROW, F), jnp.float32),
            pltpu.VMEM_SHARED((NPAD, F), jnp.float32),
            pltpu.SemaphoreType.DMA,
            pltpu.SemaphoreType.DMA,
            pltpu.SemaphoreType.DMA,
            pltpu.SemaphoreType.DMA,
        ],
    )
    def segsum(table_hbm, src_hbm, dst_hbm, zeros_hbm, out_hbm,
               srcv, dstv, rows0, rows1, acc, sg0, sg1, ss0, ss1):
        _segsum_body(F, table_hbm, src_hbm, dst_hbm, zeros_hbm, out_hbm,
                     srcv, dstv, rows0, rows1, acc, sg0, sg1, ss0, ss1)

    return segsum


_segsum16 = _make_segsum(HID)
_segsum48 = _make_segsum(CP)


def _dense1_body(x_ref, w1_ref, degp_ref, gs1_ref, dinv_ref):
    deg = 1.0 + degp_ref[0, :N, 0:1] + degp_ref[1, :N, 0:1]
    dinv = lax.rsqrt(deg)
    g = jnp.dot(x_ref[...], w1_ref[...], preferred_element_type=jnp.float32)
    gs1_ref[...] = dinv * g
    dinv_ref[...] = dinv


def _dense2_body(segp_ref, gs1_ref, dinv_ref, b1_ref, w2_ref, gs2_ref):
    dinv = dinv_ref[...]
    seg = segp_ref[0, :N] + segp_ref[1, :N]
    h1 = jnp.maximum(dinv * (seg + gs1_ref[...]) + b1_ref[...], 0.0)
    g2 = jnp.dot(h1, w2_ref[...], preferred_element_type=jnp.float32)
    gs2_ref[...] = jnp.concatenate(
        [dinv * g2, jnp.zeros((N, CP - C), jnp.float32)], axis=1)


def _dense3_body(segp_ref, gs2_ref, dinv_ref, b2_ref, out_ref):
    seg = segp_ref[0, :N, :C] + segp_ref[1, :N, :C]
    o = dinv_ref[...] * (seg + gs2_ref[:, :C]) + b2_ref[...]
    m = jnp.max(o, axis=1, keepdims=True)
    lse = jnp.log(jnp.sum(jnp.exp(o - m), axis=1, keepdims=True)) + m
    out_ref[...] = o - lse


def _dense1(x, W1, degp):
    return pl.pallas_call(
        _dense1_body,
        out_shape=(
            jax.ShapeDtypeStruct((N, HID), jnp.float32),
            jax.ShapeDtypeStruct((N, 1), jnp.float32),
        ),
    )(x, W1, degp)


def _dense2(segp, gs1, dinv, b1, W2):
    return pl.pallas_call(
        _dense2_body,
        out_shape=jax.ShapeDtypeStruct((N, CP), jnp.float32),
    )(segp, gs1, dinv, b1.reshape(1, HID), W2)


def _dense3(segp, gs2, dinv, b2):
    return pl.pallas_call(
        _dense3_body,
        out_shape=jax.ShapeDtypeStruct((N, C), jnp.float32),
    )(segp, gs2, dinv, b2.reshape(1, C))


def kernel(x, edge_index, W1, b1, W2, b2):
    src = edge_index[0].reshape(NT, NCH, K)
    dst = edge_index[1].reshape(NT, NCH, K)
    z16 = jnp.zeros((RPT, HID), jnp.float32)
    z48 = jnp.zeros((RPT, CP), jnp.float32)

    degp = _deg_pass(dst, z16)
    gs1, dinv = _dense1(x, W1, degp)
    segp1 = _segsum16(gs1, src, dst, z16)
    gs2 = _dense2(segp1, gs1, dinv, b1, W2)
    segp2 = _segsum48(gs2, src, dst, z48)
    return _dense3(segp2, gs2, dinv, b2)
